# Initial kernel scaffold; baseline (speedup 1.0000x reference)
import jax
import jax.numpy as jnp
from jax.experimental import pallas as pl


def kernel(x, edge_index, W1, b1, W2, b2, W3, b3, Wl, bl):
    raise NotImplementedError("write your pallas kernel here")


# speedup vs baseline: 5.3477x; 5.3477x over previous
import functools

import jax
import jax.numpy as jnp
from jax import lax
from jax.experimental import pallas as pl
from jax.experimental.pallas import tpu as pltpu
from jax.experimental.pallas import tpu_sc as plsc

N = 10000
NP = 10240
D = 128
E = 320000
NC = 2
NS = 16
NW = NC * NS
EPW = E // NW
KBLK = 2000
ROWS = D // NW
BL = 1280


def _mesh():
    return plsc.VectorSubcoreMesh(
        core_axis_name="c", subcore_axis_name="s",
        num_cores=NC, num_subcores=NS)


def _deg_body(dst_hbm, out_hbm, d_v, hist_v):
    wid = lax.axis_index("s") * NC + lax.axis_index("c")
    base = wid * EPW
    zeros = jnp.zeros((16,), jnp.float32)
    ones = jnp.ones((16,), jnp.float32)

    def zero_body(i, carry):
        hist_v[pl.ds(i * 16, 16)] = zeros
        return carry
    lax.fori_loop(0, NP // 16, zero_body, 0)

    def blk_body(b, carry):
        pltpu.sync_copy(dst_hbm.at[pl.ds(base + b * KBLK, KBLK)], d_v)

        def grp_body(k, c):
            dv = d_v[pl.ds(k * 16, 16)]
            plsc.addupdate_scatter(hist_v, [dv], ones)
            return c
        lax.fori_loop(0, KBLK // 16, grp_body, 0)
        return carry
    lax.fori_loop(0, EPW // KBLK, blk_body, 0)

    pltpu.sync_copy(hist_v, out_hbm.at[pl.ds(wid * NP, NP)])


@functools.cache
def _deg_kernel():
    return pl.kernel(
        _deg_body,
        out_type=jax.ShapeDtypeStruct((NW * NP,), jnp.float32),
        mesh=_mesh(),
        compiler_params=pltpu.CompilerParams(needs_layout_passes=False),
        scratch_types=[
            pltpu.VMEM((KBLK,), jnp.int32),
            pltpu.VMEM((NP,), jnp.float32),
        ],
    )


def _agg_body(g_hbm, src_hbm, dst_hbm, out_hbm, s_v, d_v, *row_refs):
    g_refs = row_refs[:ROWS]
    a_refs = row_refs[ROWS:]
    wid = lax.axis_index("s") * NC + lax.axis_index("c")
    r0 = wid * ROWS
    zeros = jnp.zeros((16,), jnp.float32)

    for c in range(ROWS):
        pltpu.sync_copy(g_hbm.at[pl.ds((r0 + c) * NP, NP)], g_refs[c])

    def zero_body(i, carry):
        for c in range(ROWS):
            a_refs[c][pl.ds(i * 16, 16)] = zeros
        return carry
    lax.fori_loop(0, NP // 16, zero_body, 0)

    def blk_body(b, carry):
        off = b * KBLK
        pltpu.sync_copy(src_hbm.at[pl.ds(off, KBLK)], s_v)
        pltpu.sync_copy(dst_hbm.at[pl.ds(off, KBLK)], d_v)

        def grp_body(k, c):
            sv = s_v[pl.ds(k * 16, 16)]
            dv = d_v[pl.ds(k * 16, 16)]
            for r in range(ROWS):
                vals = plsc.load_gather(g_refs[r], [sv])
                plsc.addupdate_scatter(a_refs[r], [dv], vals)
            return c
        lax.fori_loop(0, KBLK // 16, grp_body, 0)
        return carry
    lax.fori_loop(0, E // KBLK, blk_body, 0)

    for c in range(ROWS):
        pltpu.sync_copy(a_refs[c], out_hbm.at[pl.ds((r0 + c) * NP, NP)])


@functools.cache
def _agg_kernel():
    return pl.kernel(
        _agg_body,
        out_type=jax.ShapeDtypeStruct((D * NP,), jnp.float32),
        mesh=_mesh(),
        compiler_params=pltpu.CompilerParams(needs_layout_passes=False),
        scratch_types=[
            pltpu.VMEM((KBLK,), jnp.int32),
            pltpu.VMEM((KBLK,), jnp.int32),
        ] + [pltpu.VMEM((NP,), jnp.float32) for _ in range(2 * ROWS)],
    )


def _dis_body(hist_ref, dis_ref):
    deg = jnp.sum(hist_ref[...], axis=0, keepdims=True) + 1.0
    dis_ref[...] = lax.rsqrt(deg)


def _dis_kernel(hist):
    return pl.pallas_call(
        _dis_body,
        grid=(NP // BL,),
        in_specs=[pl.BlockSpec((NW, BL), lambda j: (0, j))],
        out_specs=pl.BlockSpec((1, BL), lambda j: (0, j)),
        out_shape=jax.ShapeDtypeStruct((1, NP), jnp.float32),
    )(hist)


def _mm_first_body(x_ref, w_ref, dis_ref, g_ref):
    h = lax.dot_general(w_ref[...], x_ref[...], (((0,), (0,)), ((), ())),
                        preferred_element_type=jnp.float32)
    g_ref[...] = h * dis_ref[...]


def _mm_first(x_t, w, dis):
    return pl.pallas_call(
        _mm_first_body,
        grid=(NP // BL,),
        in_specs=[
            pl.BlockSpec((D, BL), lambda j: (0, j)),
            pl.BlockSpec((D, D), lambda j: (0, 0)),
            pl.BlockSpec((1, BL), lambda j: (0, j)),
        ],
        out_specs=pl.BlockSpec((D, BL), lambda j: (0, j)),
        out_shape=jax.ShapeDtypeStruct((D, NP), jnp.float32),
    )(x_t, w, dis)


def _mm_mid_body(agg_ref, g_ref, dis_ref, b_ref, w_ref, out_ref, *, relu):
    dis = dis_ref[...]
    x = dis * (agg_ref[...] + g_ref[...]) + b_ref[...]
    if relu:
        x = jnp.maximum(x, 0.0)
    h = lax.dot_general(w_ref[...], x, (((0,), (0,)), ((), ())),
                        preferred_element_type=jnp.float32)
    out_ref[...] = h * dis


def _mm_mid(agg, g, dis, b_col, w_next, relu):
    return pl.pallas_call(
        functools.partial(_mm_mid_body, relu=relu),
        grid=(NP // BL,),
        in_specs=[
            pl.BlockSpec((D, BL), lambda j: (0, j)),
            pl.BlockSpec((D, BL), lambda j: (0, j)),
            pl.BlockSpec((1, BL), lambda j: (0, j)),
            pl.BlockSpec((D, 1), lambda j: (0, 0)),
            pl.BlockSpec((D, D), lambda j: (0, 0)),
        ],
        out_specs=pl.BlockSpec((D, BL), lambda j: (0, j)),
        out_shape=jax.ShapeDtypeStruct((D, NP), jnp.float32),
    )(agg, g, dis, b_col, w_next)


def _mm_final_body(agg_ref, g_ref, dis_ref, b_ref, wl_ref, bl_ref, out_ref):
    x = dis_ref[...] * (agg_ref[...] + g_ref[...]) + b_ref[...]
    out_ref[...] = lax.dot_general(
        x, wl_ref[...], (((0,), (0,)), ((), ())),
        preferred_element_type=jnp.float32) + bl_ref[...]


def _mm_final(agg, g, dis, b_col, wl, bl_row):
    c = wl.shape[1]
    return pl.pallas_call(
        _mm_final_body,
        grid=(NP // BL,),
        in_specs=[
            pl.BlockSpec((D, BL), lambda j: (0, j)),
            pl.BlockSpec((D, BL), lambda j: (0, j)),
            pl.BlockSpec((1, BL), lambda j: (0, j)),
            pl.BlockSpec((D, 1), lambda j: (0, 0)),
            pl.BlockSpec((D, c), lambda j: (0, 0)),
            pl.BlockSpec((1, c), lambda j: (0, 0)),
        ],
        out_specs=pl.BlockSpec((BL, c), lambda j: (j, 0)),
        out_shape=jax.ShapeDtypeStruct((NP, c), jnp.float32),
    )(agg, g, dis, b_col, wl, bl_row)


def kernel(x, edge_index, W1, b1, W2, b2, W3, b3, Wl, bl):
    src = edge_index[0].astype(jnp.int32)
    dst = edge_index[1].astype(jnp.int32)

    x_t = jnp.pad(x.T, ((0, 0), (0, NP - N)))

    hist = _deg_kernel()(dst).reshape(NW, NP)
    dis = _dis_kernel(hist)

    agg = _agg_kernel()
    g1 = _mm_first(x_t, W1, dis)
    a1 = agg(g1.reshape(-1), src, dst).reshape(D, NP)
    g2 = _mm_mid(a1, g1, dis, b1.reshape(D, 1), W2, relu=True)
    a2 = agg(g2.reshape(-1), src, dst).reshape(D, NP)
    g3 = _mm_mid(a2, g2, dis, b2.reshape(D, 1), W3, relu=True)
    a3 = agg(g3.reshape(-1), src, dst).reshape(D, NP)
    out = _mm_final(a3, g3, dis, b3.reshape(D, 1), Wl, bl.reshape(1, -1))
    return out[:N]


# speedup vs baseline: 6.8107x; 1.2736x over previous
import functools

import jax
import jax.numpy as jnp
from jax import lax
from jax.experimental import pallas as pl
from jax.experimental.pallas import tpu as pltpu
from jax.experimental.pallas import tpu_sc as plsc

N = 10000
NP = 10240
D = 128
E = 320000
NC = 2
NS = 16
NW = NC * NS
EPW = E // NW
KBLK = 2000
ABLK = 20000
UNROLL = 5
ROWS = D // NW
BL = 1280


def _mesh():
    return plsc.VectorSubcoreMesh(
        core_axis_name="c", subcore_axis_name="s",
        num_cores=NC, num_subcores=NS)


def _deg_body(dst_hbm, out_hbm, d_v, hist_v):
    wid = lax.axis_index("s") * NC + lax.axis_index("c")
    base = wid * EPW
    zeros = jnp.zeros((16,), jnp.float32)
    ones = jnp.ones((16,), jnp.float32)

    def zero_body(i, carry):
        hist_v[pl.ds(i * 16, 16)] = zeros
        return carry
    lax.fori_loop(0, NP // 16, zero_body, 0)

    def blk_body(b, carry):
        pltpu.sync_copy(dst_hbm.at[pl.ds(base + b * KBLK, KBLK)], d_v)

        def grp_body(k, c):
            dv = d_v[pl.ds(k * 16, 16)]
            plsc.addupdate_scatter(hist_v, [dv], ones)
            return c
        lax.fori_loop(0, KBLK // 16, grp_body, 0)
        return carry
    lax.fori_loop(0, EPW // KBLK, blk_body, 0)

    pltpu.sync_copy(hist_v, out_hbm.at[pl.ds(wid * NP, NP)])


@functools.cache
def _deg_kernel():
    return pl.kernel(
        _deg_body,
        out_type=jax.ShapeDtypeStruct((NW * NP,), jnp.float32),
        mesh=_mesh(),
        compiler_params=pltpu.CompilerParams(needs_layout_passes=False),
        scratch_types=[
            pltpu.VMEM((KBLK,), jnp.int32),
            pltpu.VMEM((NP,), jnp.float32),
        ],
    )


def _agg_body(g_hbm, src_hbm, dst_hbm, out_hbm, s_v, d_v, *row_refs):
    g_refs = row_refs[:ROWS]
    a_refs = row_refs[ROWS:]
    wid = lax.axis_index("s") * NC + lax.axis_index("c")
    r0 = wid * ROWS
    zeros = jnp.zeros((16,), jnp.float32)

    for c in range(ROWS):
        pltpu.sync_copy(g_hbm.at[pl.ds((r0 + c) * NP, NP)], g_refs[c])

    def zero_body(i, carry):
        for c in range(ROWS):
            a_refs[c][pl.ds(i * 16, 16)] = zeros
        return carry
    lax.fori_loop(0, NP // 16, zero_body, 0)

    def blk_body(b, carry):
        off = b * ABLK
        pltpu.sync_copy(src_hbm.at[pl.ds(off, ABLK)], s_v)
        pltpu.sync_copy(dst_hbm.at[pl.ds(off, ABLK)], d_v)

        def grp_body(k, c):
            for u in range(UNROLL):
                sv = s_v[pl.ds((k * UNROLL + u) * 16, 16)]
                dv = d_v[pl.ds((k * UNROLL + u) * 16, 16)]
                for r in range(ROWS):
                    vals = plsc.load_gather(g_refs[r], [sv])
                    plsc.addupdate_scatter(a_refs[r], [dv], vals)
            return c
        lax.fori_loop(0, ABLK // (16 * UNROLL), grp_body, 0)
        return carry
    lax.fori_loop(0, E // ABLK, blk_body, 0)

    for c in range(ROWS):
        pltpu.sync_copy(a_refs[c], out_hbm.at[pl.ds((r0 + c) * NP, NP)])


@functools.cache
def _agg_kernel():
    return pl.kernel(
        _agg_body,
        out_type=jax.ShapeDtypeStruct((D * NP,), jnp.float32),
        mesh=_mesh(),
        compiler_params=pltpu.CompilerParams(needs_layout_passes=False),
        scratch_types=[
            pltpu.VMEM((ABLK,), jnp.int32),
            pltpu.VMEM((ABLK,), jnp.int32),
        ] + [pltpu.VMEM((NP,), jnp.float32) for _ in range(2 * ROWS)],
    )


def _dis_body(hist_ref, dis_ref):
    deg = jnp.sum(hist_ref[...], axis=0, keepdims=True) + 1.0
    dis_ref[...] = lax.rsqrt(deg)


def _dis_kernel(hist):
    return pl.pallas_call(
        _dis_body,
        grid=(NP // BL,),
        in_specs=[pl.BlockSpec((NW, BL), lambda j: (0, j))],
        out_specs=pl.BlockSpec((1, BL), lambda j: (0, j)),
        out_shape=jax.ShapeDtypeStruct((1, NP), jnp.float32),
    )(hist)


def _mm_first_body(x_ref, w_ref, dis_ref, g_ref):
    h = lax.dot_general(w_ref[...], x_ref[...], (((0,), (0,)), ((), ())),
                        preferred_element_type=jnp.float32)
    g_ref[...] = h * dis_ref[...]


def _mm_first(x_t, w, dis):
    return pl.pallas_call(
        _mm_first_body,
        grid=(NP // BL,),
        in_specs=[
            pl.BlockSpec((D, BL), lambda j: (0, j)),
            pl.BlockSpec((D, D), lambda j: (0, 0)),
            pl.BlockSpec((1, BL), lambda j: (0, j)),
        ],
        out_specs=pl.BlockSpec((D, BL), lambda j: (0, j)),
        out_shape=jax.ShapeDtypeStruct((D, NP), jnp.float32),
    )(x_t, w, dis)


def _mm_mid_body(agg_ref, g_ref, dis_ref, b_ref, w_ref, out_ref, *, relu):
    dis = dis_ref[...]
    x = dis * (agg_ref[...] + g_ref[...]) + b_ref[...]
    if relu:
        x = jnp.maximum(x, 0.0)
    h = lax.dot_general(w_ref[...], x, (((0,), (0,)), ((), ())),
                        preferred_element_type=jnp.float32)
    out_ref[...] = h * dis


def _mm_mid(agg, g, dis, b_col, w_next, relu):
    return pl.pallas_call(
        functools.partial(_mm_mid_body, relu=relu),
        grid=(NP // BL,),
        in_specs=[
            pl.BlockSpec((D, BL), lambda j: (0, j)),
            pl.BlockSpec((D, BL), lambda j: (0, j)),
            pl.BlockSpec((1, BL), lambda j: (0, j)),
            pl.BlockSpec((D, 1), lambda j: (0, 0)),
            pl.BlockSpec((D, D), lambda j: (0, 0)),
        ],
        out_specs=pl.BlockSpec((D, BL), lambda j: (0, j)),
        out_shape=jax.ShapeDtypeStruct((D, NP), jnp.float32),
    )(agg, g, dis, b_col, w_next)


def _mm_final_body(agg_ref, g_ref, dis_ref, b_ref, wl_ref, bl_ref, out_ref):
    x = dis_ref[...] * (agg_ref[...] + g_ref[...]) + b_ref[...]
    out_ref[...] = lax.dot_general(
        x, wl_ref[...], (((0,), (0,)), ((), ())),
        preferred_element_type=jnp.float32) + bl_ref[...]


def _mm_final(agg, g, dis, b_col, wl, bl_row):
    c = wl.shape[1]
    return pl.pallas_call(
        _mm_final_body,
        grid=(NP // BL,),
        in_specs=[
            pl.BlockSpec((D, BL), lambda j: (0, j)),
            pl.BlockSpec((D, BL), lambda j: (0, j)),
            pl.BlockSpec((1, BL), lambda j: (0, j)),
            pl.BlockSpec((D, 1), lambda j: (0, 0)),
            pl.BlockSpec((D, c), lambda j: (0, 0)),
            pl.BlockSpec((1, c), lambda j: (0, 0)),
        ],
        out_specs=pl.BlockSpec((BL, c), lambda j: (j, 0)),
        out_shape=jax.ShapeDtypeStruct((NP, c), jnp.float32),
    )(agg, g, dis, b_col, wl, bl_row)


def kernel(x, edge_index, W1, b1, W2, b2, W3, b3, Wl, bl):
    src = edge_index[0].astype(jnp.int32)
    dst = edge_index[1].astype(jnp.int32)

    x_t = jnp.pad(x.T, ((0, 0), (0, NP - N)))

    hist = _deg_kernel()(dst).reshape(NW, NP)
    dis = _dis_kernel(hist)

    agg = _agg_kernel()
    g1 = _mm_first(x_t, W1, dis)
    a1 = agg(g1.reshape(-1), src, dst).reshape(D, NP)
    g2 = _mm_mid(a1, g1, dis, b1.reshape(D, 1), W2, relu=True)
    a2 = agg(g2.reshape(-1), src, dst).reshape(D, NP)
    g3 = _mm_mid(a2, g2, dis, b2.reshape(D, 1), W3, relu=True)
    a3 = agg(g3.reshape(-1), src, dst).reshape(D, NP)
    out = _mm_final(a3, g3, dis, b3.reshape(D, 1), Wl, bl.reshape(1, -1))
    return out[:N]


# speedup vs baseline: 7.3625x; 1.0810x over previous
import functools

import jax
import jax.numpy as jnp
from jax import lax
from jax.experimental import pallas as pl
from jax.experimental.pallas import tpu as pltpu
from jax.experimental.pallas import tpu_sc as plsc

N = 10000
NP = 10240
D = 128
E = 320000
NC = 2
NS = 16
NW = NC * NS
EPW = E // NW
KBLK = 2000
G = 128
JROWS = 80
EPA = JROWS * G
EP = NW * EPA
NPT = NP // NS
BLR = 1280


def _mesh():
    return plsc.VectorSubcoreMesh(
        core_axis_name="c", subcore_axis_name="s",
        num_cores=NC, num_subcores=NS)


def _deg_body(dst_hbm, out_hbm, d_v, hist_v):
    wid = lax.axis_index("s") * NC + lax.axis_index("c")
    base = wid * EPW
    zeros = jnp.zeros((16,), jnp.float32)
    ones = jnp.ones((16,), jnp.float32)

    def zero_body(i, carry):
        hist_v[pl.ds(i * 16, 16)] = zeros
        return carry
    lax.fori_loop(0, NP // 16, zero_body, 0)

    def blk_body(b, carry):
        pltpu.sync_copy(dst_hbm.at[pl.ds(base + b * KBLK, KBLK)], d_v)

        def grp_body(k, c):
            dv = d_v[pl.ds(k * 16, 16)]
            plsc.addupdate_scatter(hist_v, [dv], ones)
            return c
        lax.fori_loop(0, KBLK // 16, grp_body, 0)
        return carry
    lax.fori_loop(0, EPW // KBLK, blk_body, 0)

    pltpu.sync_copy(hist_v, out_hbm.at[pl.ds(wid * NP, NP)])


@functools.cache
def _deg_kernel():
    return pl.kernel(
        _deg_body,
        out_type=jax.ShapeDtypeStruct((NW * NP,), jnp.float32),
        mesh=_mesh(),
        compiler_params=pltpu.CompilerParams(needs_layout_passes=False),
        scratch_types=[
            pltpu.VMEM((KBLK,), jnp.int32),
            pltpu.VMEM((NP,), jnp.float32),
        ],
    )


def _agg_body(g_hbm, src_hbm, dst_hbm, z_hbm, out_hbm,
              s_idx, d_idx, rows_v, acc, sem):
    cid = lax.axis_index("c")
    sid = lax.axis_index("s")
    w2 = cid * NS + sid
    base = w2 * JROWS

    pltpu.sync_copy(z_hbm, acc.at[pl.ds(sid * NPT, NPT)])
    pltpu.sync_copy(src_hbm.at[pl.ds(base, JROWS)], s_idx)
    pltpu.sync_copy(dst_hbm.at[pl.ds(base, JROWS)], d_idx)
    plsc.subcore_barrier()

    def blk_body(j, carry):
        pltpu.async_copy(g_hbm.at[s_idx.at[j]], rows_v, sem).wait()
        pltpu.sync_copy(rows_v, acc.at[d_idx.at[j]], add=True)
        return carry
    lax.fori_loop(0, JROWS, blk_body, 0)

    plsc.subcore_barrier()
    pltpu.sync_copy(acc.at[pl.ds(sid * NPT, NPT)],
                    out_hbm.at[pl.ds(cid * NP + sid * NPT, NPT)])


@functools.cache
def _agg_kernel():
    return pl.kernel(
        _agg_body,
        out_type=jax.ShapeDtypeStruct((NC * NP, D), jnp.float32),
        mesh=_mesh(),
        compiler_params=pltpu.CompilerParams(needs_layout_passes=False),
        scratch_types=[
            pltpu.VMEM((JROWS, G), jnp.int32),
            pltpu.VMEM((JROWS, G), jnp.int32),
            pltpu.VMEM((G, D), jnp.float32),
            pltpu.VMEM_SHARED((NP, D), jnp.float32),
            pltpu.SemaphoreType.DMA,
        ],
    )


def _dis_body(hist_ref, dis_ref):
    deg = jnp.sum(hist_ref[...], axis=0, keepdims=True) + 1.0
    dis_ref[...] = lax.rsqrt(deg)


def _dis_kernel(hist):
    return pl.pallas_call(
        _dis_body,
        grid=(NP // BLR,),
        in_specs=[pl.BlockSpec((NW, BLR), lambda j: (0, j))],
        out_specs=pl.BlockSpec((1, BLR), lambda j: (0, j)),
        out_shape=jax.ShapeDtypeStruct((1, NP), jnp.float32),
    )(hist)


def _mm_first_body(x_ref, w_ref, dis_ref, g_ref):
    h = lax.dot_general(x_ref[...], w_ref[...], (((1,), (0,)), ((), ())),
                        preferred_element_type=jnp.float32)
    g_ref[...] = h * dis_ref[...]


def _mm_first(x_p, w, dis_col):
    return pl.pallas_call(
        _mm_first_body,
        grid=(NP // BLR,),
        in_specs=[
            pl.BlockSpec((BLR, D), lambda j: (j, 0)),
            pl.BlockSpec((D, D), lambda j: (0, 0)),
            pl.BlockSpec((BLR, 1), lambda j: (j, 0)),
        ],
        out_specs=pl.BlockSpec((BLR, D), lambda j: (j, 0)),
        out_shape=jax.ShapeDtypeStruct((NP, D), jnp.float32),
    )(x_p, w, dis_col)


def _mm_mid_body(p0_ref, p1_ref, g_ref, dis_ref, b_ref, w_ref, out_ref, *,
                 relu):
    dis = dis_ref[...]
    x = dis * (p0_ref[...] + p1_ref[...] + g_ref[...]) + b_ref[...]
    if relu:
        x = jnp.maximum(x, 0.0)
    h = lax.dot_general(x, w_ref[...], (((1,), (0,)), ((), ())),
                        preferred_element_type=jnp.float32)
    out_ref[...] = h * dis


def _mm_mid(p, g, dis_col, b_row, w_next, relu):
    return pl.pallas_call(
        functools.partial(_mm_mid_body, relu=relu),
        grid=(NP // BLR,),
        in_specs=[
            pl.BlockSpec((BLR, D), lambda j: (j, 0)),
            pl.BlockSpec((BLR, D), lambda j: (NP // BLR + j, 0)),
            pl.BlockSpec((BLR, D), lambda j: (j, 0)),
            pl.BlockSpec((BLR, 1), lambda j: (j, 0)),
            pl.BlockSpec((1, D), lambda j: (0, 0)),
            pl.BlockSpec((D, D), lambda j: (0, 0)),
        ],
        out_specs=pl.BlockSpec((BLR, D), lambda j: (j, 0)),
        out_shape=jax.ShapeDtypeStruct((NP, D), jnp.float32),
    )(p, p, g, dis_col, b_row, w_next)


def _mm_final_body(p0_ref, p1_ref, g_ref, dis_ref, b_ref, wl_ref, bl_ref,
                   out_ref):
    x = dis_ref[...] * (p0_ref[...] + p1_ref[...] + g_ref[...]) + b_ref[...]
    out_ref[...] = lax.dot_general(
        x, wl_ref[...], (((1,), (0,)), ((), ())),
        preferred_element_type=jnp.float32) + bl_ref[...]


def _mm_final(p, g, dis_col, b_row, wl, bl_row):
    c = wl.shape[1]
    return pl.pallas_call(
        _mm_final_body,
        grid=(NP // BLR,),
        in_specs=[
            pl.BlockSpec((BLR, D), lambda j: (j, 0)),
            pl.BlockSpec((BLR, D), lambda j: (NP // BLR + j, 0)),
            pl.BlockSpec((BLR, D), lambda j: (j, 0)),
            pl.BlockSpec((BLR, 1), lambda j: (j, 0)),
            pl.BlockSpec((1, D), lambda j: (0, 0)),
            pl.BlockSpec((D, c), lambda j: (0, 0)),
            pl.BlockSpec((1, c), lambda j: (0, 0)),
        ],
        out_specs=pl.BlockSpec((BLR, c), lambda j: (j, 0)),
        out_shape=jax.ShapeDtypeStruct((NP, c), jnp.float32),
    )(p, p, g, dis_col, b_row, wl, bl_row)


def kernel(x, edge_index, W1, b1, W2, b2, W3, b3, Wl, bl):
    src = edge_index[0].astype(jnp.int32)
    dst = edge_index[1].astype(jnp.int32)
    pad = jnp.full((EP - E,), N, jnp.int32)
    src_p = jnp.concatenate([src, pad]).reshape(NW * JROWS, G)
    dst_p = jnp.concatenate([dst, pad]).reshape(NW * JROWS, G)

    x_p = jnp.pad(x, ((0, NP - N), (0, 0)))
    zeros_tile = jnp.zeros((NPT, D), jnp.float32)

    hist = _deg_kernel()(dst).reshape(NW, NP)
    dis_col = _dis_kernel(hist).reshape(NP, 1)

    agg = _agg_kernel()
    g1 = _mm_first(x_p, W1, dis_col)
    p1 = agg(g1, src_p, dst_p, zeros_tile)
    g2 = _mm_mid(p1, g1, dis_col, b1.reshape(1, D), W2, relu=True)
    p2 = agg(g2, src_p, dst_p, zeros_tile)
    g3 = _mm_mid(p2, g2, dis_col, b2.reshape(1, D), W3, relu=True)
    p3 = agg(g3, src_p, dst_p, zeros_tile)
    out = _mm_final(p3, g3, dis_col, b3.reshape(1, D), Wl, bl.reshape(1, -1))
    return out[:N]


# speedup vs baseline: 8.0348x; 1.0913x over previous
import functools

import jax
import jax.numpy as jnp
from jax import lax
from jax.experimental import pallas as pl
from jax.experimental.pallas import tpu as pltpu
from jax.experimental.pallas import tpu_sc as plsc

N = 10000
NP = 10240
D = 128
E = 320000
NC = 2
NS = 16
NW = NC * NS
EPW = E // NW
KBLK = 2000
G = 128
JROWS = 80
CHROWS = 16
EPA = JROWS * G
EP = NW * EPA
NPT = NP // NS
BLR = 1280


def _mesh():
    return plsc.VectorSubcoreMesh(
        core_axis_name="c", subcore_axis_name="s",
        num_cores=NC, num_subcores=NS)


def _deg_body(dst_hbm, out_hbm, d_v, hist_v):
    wid = lax.axis_index("s") * NC + lax.axis_index("c")
    base = wid * EPW
    zeros = jnp.zeros((16,), jnp.float32)
    ones = jnp.ones((16,), jnp.float32)

    def zero_body(i, carry):
        hist_v[pl.ds(i * 16, 16)] = zeros
        return carry
    lax.fori_loop(0, NP // 16, zero_body, 0)

    def blk_body(b, carry):
        pltpu.sync_copy(dst_hbm.at[pl.ds(base + b * KBLK, KBLK)], d_v)

        def grp_body(k, c):
            dv = d_v[pl.ds(k * 16, 16)]
            plsc.addupdate_scatter(hist_v, [dv], ones)
            return c
        lax.fori_loop(0, KBLK // 16, grp_body, 0)
        return carry
    lax.fori_loop(0, EPW // KBLK, blk_body, 0)

    pltpu.sync_copy(hist_v, out_hbm.at[pl.ds(wid * NP, NP)])


@functools.cache
def _deg_kernel():
    return pl.kernel(
        _deg_body,
        out_type=jax.ShapeDtypeStruct((NW * NP,), jnp.float32),
        mesh=_mesh(),
        compiler_params=pltpu.CompilerParams(needs_layout_passes=False),
        scratch_types=[
            pltpu.VMEM((KBLK,), jnp.int32),
            pltpu.VMEM((NP,), jnp.float32),
        ],
    )


def _agg_body(g_hbm, src_hbm, dst_hbm, z_hbm, out_hbm,
              s_idx, d_idx, rows_a, rows_b, acc, sem_a, sem_b):
    cid = lax.axis_index("c")
    sid = lax.axis_index("s")
    w2 = cid * NS + sid
    base = w2 * JROWS

    pltpu.sync_copy(z_hbm, acc.at[pl.ds(sid * NPT, NPT)])
    plsc.subcore_barrier()

    bufs = (rows_a, rows_b)
    sems = (sem_a, sem_b)
    copies = [None, None]
    for c in range(JROWS // CHROWS):
        pltpu.sync_copy(src_hbm.at[pl.ds(base + c * CHROWS, CHROWS)], s_idx)
        pltpu.sync_copy(dst_hbm.at[pl.ds(base + c * CHROWS, CHROWS)], d_idx)
        copies[0] = pltpu.async_copy(g_hbm.at[s_idx.at[0]], rows_a, sem_a)
        for j in range(CHROWS):
            cur = j % 2
            nxt = (j + 1) % 2
            if j + 1 < CHROWS:
                copies[nxt] = pltpu.async_copy(
                    g_hbm.at[s_idx.at[j + 1]], bufs[nxt], sems[nxt])
            copies[cur].wait()
            pltpu.sync_copy(bufs[cur], acc.at[d_idx.at[j]], add=True)

    plsc.subcore_barrier()
    pltpu.sync_copy(acc.at[pl.ds(sid * NPT, NPT)],
                    out_hbm.at[pl.ds(cid * NP + sid * NPT, NPT)])


@functools.cache
def _agg_kernel():
    return pl.kernel(
        _agg_body,
        out_type=jax.ShapeDtypeStruct((NC * NP, D), jnp.float32),
        mesh=_mesh(),
        compiler_params=pltpu.CompilerParams(needs_layout_passes=False),
        scratch_types=[
            pltpu.VMEM((CHROWS, G), jnp.int32),
            pltpu.VMEM((CHROWS, G), jnp.int32),
            pltpu.VMEM((G, D), jnp.float32),
            pltpu.VMEM((G, D), jnp.float32),
            pltpu.VMEM_SHARED((NP, D), jnp.float32),
            pltpu.SemaphoreType.DMA,
            pltpu.SemaphoreType.DMA,
        ],
    )


def _dis_body(hist_ref, dis_ref):
    deg = jnp.sum(hist_ref[...], axis=0, keepdims=True) + 1.0
    dis_ref[...] = lax.rsqrt(deg)


def _dis_kernel(hist):
    return pl.pallas_call(
        _dis_body,
        grid=(NP // BLR,),
        in_specs=[pl.BlockSpec((NW, BLR), lambda j: (0, j))],
        out_specs=pl.BlockSpec((1, BLR), lambda j: (0, j)),
        out_shape=jax.ShapeDtypeStruct((1, NP), jnp.float32),
    )(hist)


def _mm_first_body(x_ref, w_ref, dis_ref, g_ref):
    h = lax.dot_general(x_ref[...], w_ref[...], (((1,), (0,)), ((), ())),
                        preferred_element_type=jnp.float32)
    g_ref[...] = h * dis_ref[...]


def _mm_first(x_p, w, dis_col):
    return pl.pallas_call(
        _mm_first_body,
        grid=(NP // BLR,),
        in_specs=[
            pl.BlockSpec((BLR, D), lambda j: (j, 0)),
            pl.BlockSpec((D, D), lambda j: (0, 0)),
            pl.BlockSpec((BLR, 1), lambda j: (j, 0)),
        ],
        out_specs=pl.BlockSpec((BLR, D), lambda j: (j, 0)),
        out_shape=jax.ShapeDtypeStruct((NP, D), jnp.float32),
    )(x_p, w, dis_col)


def _mm_mid_body(p0_ref, p1_ref, g_ref, dis_ref, b_ref, w_ref, out_ref, *,
                 relu):
    dis = dis_ref[...]
    x = dis * (p0_ref[...] + p1_ref[...] + g_ref[...]) + b_ref[...]
    if relu:
        x = jnp.maximum(x, 0.0)
    h = lax.dot_general(x, w_ref[...], (((1,), (0,)), ((), ())),
                        preferred_element_type=jnp.float32)
    out_ref[...] = h * dis


def _mm_mid(p, g, dis_col, b_row, w_next, relu):
    return pl.pallas_call(
        functools.partial(_mm_mid_body, relu=relu),
        grid=(NP // BLR,),
        in_specs=[
            pl.BlockSpec((BLR, D), lambda j: (j, 0)),
            pl.BlockSpec((BLR, D), lambda j: (NP // BLR + j, 0)),
            pl.BlockSpec((BLR, D), lambda j: (j, 0)),
            pl.BlockSpec((BLR, 1), lambda j: (j, 0)),
            pl.BlockSpec((1, D), lambda j: (0, 0)),
            pl.BlockSpec((D, D), lambda j: (0, 0)),
        ],
        out_specs=pl.BlockSpec((BLR, D), lambda j: (j, 0)),
        out_shape=jax.ShapeDtypeStruct((NP, D), jnp.float32),
    )(p, p, g, dis_col, b_row, w_next)


def _mm_final_body(p0_ref, p1_ref, g_ref, dis_ref, b_ref, wl_ref, bl_ref,
                   out_ref):
    x = dis_ref[...] * (p0_ref[...] + p1_ref[...] + g_ref[...]) + b_ref[...]
    out_ref[...] = lax.dot_general(
        x, wl_ref[...], (((1,), (0,)), ((), ())),
        preferred_element_type=jnp.float32) + bl_ref[...]


def _mm_final(p, g, dis_col, b_row, wl, bl_row):
    c = wl.shape[1]
    return pl.pallas_call(
        _mm_final_body,
        grid=(NP // BLR,),
        in_specs=[
            pl.BlockSpec((BLR, D), lambda j: (j, 0)),
            pl.BlockSpec((BLR, D), lambda j: (NP // BLR + j, 0)),
            pl.BlockSpec((BLR, D), lambda j: (j, 0)),
            pl.BlockSpec((BLR, 1), lambda j: (j, 0)),
            pl.BlockSpec((1, D), lambda j: (0, 0)),
            pl.BlockSpec((D, c), lambda j: (0, 0)),
            pl.BlockSpec((1, c), lambda j: (0, 0)),
        ],
        out_specs=pl.BlockSpec((BLR, c), lambda j: (j, 0)),
        out_shape=jax.ShapeDtypeStruct((NP, c), jnp.float32),
    )(p, p, g, dis_col, b_row, wl, bl_row)


def kernel(x, edge_index, W1, b1, W2, b2, W3, b3, Wl, bl):
    src = edge_index[0].astype(jnp.int32)
    dst = edge_index[1].astype(jnp.int32)
    pad = jnp.full((EP - E,), N, jnp.int32)
    src_p = jnp.concatenate([src, pad]).reshape(NW * JROWS, G)
    dst_p = jnp.concatenate([dst, pad]).reshape(NW * JROWS, G)

    x_p = jnp.pad(x, ((0, NP - N), (0, 0)))
    zeros_tile = jnp.zeros((NPT, D), jnp.float32)

    hist = _deg_kernel()(dst).reshape(NW, NP)
    dis_col = _dis_kernel(hist).reshape(NP, 1)

    agg = _agg_kernel()
    g1 = _mm_first(x_p, W1, dis_col)
    p1 = agg(g1, src_p, dst_p, zeros_tile)
    g2 = _mm_mid(p1, g1, dis_col, b1.reshape(1, D), W2, relu=True)
    p2 = agg(g2, src_p, dst_p, zeros_tile)
    g3 = _mm_mid(p2, g2, dis_col, b2.reshape(1, D), W3, relu=True)
    p3 = agg(g3, src_p, dst_p, zeros_tile)
    out = _mm_final(p3, g3, dis_col, b3.reshape(1, D), Wl, bl.reshape(1, -1))
    return out[:N]


# speedup vs baseline: 25.7909x; 3.2099x over previous
import functools

import jax
import jax.numpy as jnp
from jax import lax
from jax.experimental import pallas as pl
from jax.experimental.pallas import tpu as pltpu
from jax.experimental.pallas import tpu_sc as plsc

N = 10000
NP = 10240
D = 128
E = 320000
NC = 2
NS = 16
NW = NC * NS
EPW = E // NW
KBLK = 2000
G = 128
JROWS = 80
CHROWS = 16
EPA = JROWS * G
EP = NW * EPA
NPT = NP // NS
BLR = 1280


def _mesh():
    return plsc.VectorSubcoreMesh(
        core_axis_name="c", subcore_axis_name="s",
        num_cores=NC, num_subcores=NS)


def _deg_body(dst_hbm, out_hbm, d_v, hist_v):
    wid = lax.axis_index("s") * NC + lax.axis_index("c")
    base = wid * EPW
    zeros = jnp.zeros((16,), jnp.float32)
    ones = jnp.ones((16,), jnp.float32)

    def zero_body(i, carry):
        hist_v[pl.ds(i * 16, 16)] = zeros
        return carry
    lax.fori_loop(0, NP // 16, zero_body, 0)

    def blk_body(b, carry):
        pltpu.sync_copy(dst_hbm.at[pl.ds(base + b * KBLK, KBLK)], d_v)

        def grp_body(k, c):
            dv = d_v[pl.ds(k * 16, 16)]
            plsc.addupdate_scatter(hist_v, [dv], ones)
            return c
        lax.fori_loop(0, KBLK // 16, grp_body, 0)
        return carry
    lax.fori_loop(0, EPW // KBLK, blk_body, 0)

    pltpu.sync_copy(hist_v, out_hbm.at[pl.ds(wid * NP, NP)])


@functools.cache
def _deg_kernel():
    return pl.kernel(
        _deg_body,
        out_type=jax.ShapeDtypeStruct((NW * NP,), jnp.float32),
        mesh=_mesh(),
        compiler_params=pltpu.CompilerParams(needs_layout_passes=False),
        scratch_types=[
            pltpu.VMEM((KBLK,), jnp.int32),
            pltpu.VMEM((NP,), jnp.float32),
        ],
    )


def _agg_body(g_hbm, src_hbm, dst_hbm, z_hbm, out_hbm,
              s_idx, d_idx, rows_a, rows_b, acc, sem_a, sem_b):
    cid = lax.axis_index("c")
    sid = lax.axis_index("s")
    w2 = cid * NS + sid
    base = w2 * JROWS

    pltpu.sync_copy(z_hbm, acc.at[pl.ds(sid * NPT, NPT)])
    plsc.subcore_barrier()

    bufs = (rows_a, rows_b)
    sems = (sem_a, sem_b)
    copies = [None, None]
    for c in range(JROWS // CHROWS):
        pltpu.sync_copy(src_hbm.at[pl.ds(base + c * CHROWS, CHROWS)], s_idx)
        pltpu.sync_copy(dst_hbm.at[pl.ds(base + c * CHROWS, CHROWS)], d_idx)
        copies[0] = pltpu.async_copy(g_hbm.at[s_idx.at[0]], rows_a, sem_a)
        for j in range(CHROWS):
            cur = j % 2
            nxt = (j + 1) % 2
            if j + 1 < CHROWS:
                copies[nxt] = pltpu.async_copy(
                    g_hbm.at[s_idx.at[j + 1]], bufs[nxt], sems[nxt])
            copies[cur].wait()
            pltpu.sync_copy(bufs[cur], acc.at[d_idx.at[j]], add=True)

    plsc.subcore_barrier()
    pltpu.sync_copy(acc.at[pl.ds(sid * NPT, NPT)],
                    out_hbm.at[pl.ds(cid * NP + sid * NPT, NPT)])


@functools.cache
def _agg_kernel():
    return pl.kernel(
        _agg_body,
        out_type=jax.ShapeDtypeStruct((NC * NP, D), jnp.float32),
        mesh=_mesh(),
        compiler_params=pltpu.CompilerParams(needs_layout_passes=False),
        scratch_types=[
            pltpu.VMEM((CHROWS, G), jnp.int32),
            pltpu.VMEM((CHROWS, G), jnp.int32),
            pltpu.VMEM((G, D), jnp.float32),
            pltpu.VMEM((G, D), jnp.float32),
            pltpu.VMEM_SHARED((NP, D), jnp.float32),
            pltpu.SemaphoreType.DMA,
            pltpu.SemaphoreType.DMA,
        ],
    )


def _dis_body(hist_ref, dis_ref):
    deg = jnp.sum(hist_ref[...], axis=0, keepdims=True) + 1.0
    dis_ref[...] = lax.rsqrt(deg)


def _dis_kernel(hist):
    return pl.pallas_call(
        _dis_body,
        grid=(NP // BLR,),
        in_specs=[pl.BlockSpec((NW, BLR), lambda j: (0, j))],
        out_specs=pl.BlockSpec((1, BLR), lambda j: (0, j)),
        out_shape=jax.ShapeDtypeStruct((1, NP), jnp.float32),
    )(hist)


def _mm_first_body(x_ref, w_ref, dis_ref, g_ref):
    h = lax.dot_general(x_ref[...], w_ref[...], (((1,), (0,)), ((), ())),
                        preferred_element_type=jnp.float32)
    g_ref[...] = h * dis_ref[...]


def _mm_first(x_p, w, dis_col):
    return pl.pallas_call(
        _mm_first_body,
        grid=(NP // BLR,),
        in_specs=[
            pl.BlockSpec((BLR, D), lambda j: (j, 0)),
            pl.BlockSpec((D, D), lambda j: (0, 0)),
            pl.BlockSpec((BLR, 1), lambda j: (j, 0)),
        ],
        out_specs=pl.BlockSpec((BLR, D), lambda j: (j, 0)),
        out_shape=jax.ShapeDtypeStruct((NP, D), jnp.float32),
    )(x_p, w, dis_col)


def _mm_mid_body(p0_ref, p1_ref, g_ref, dis_ref, b_ref, w_ref, out_ref, *,
                 relu):
    dis = dis_ref[...]
    x = dis * (p0_ref[...] + p1_ref[...] + g_ref[...]) + b_ref[...]
    if relu:
        x = jnp.maximum(x, 0.0)
    h = lax.dot_general(x, w_ref[...], (((1,), (0,)), ((), ())),
                        preferred_element_type=jnp.float32)
    out_ref[...] = h * dis


def _mm_mid(p, g, dis_col, b_row, w_next, relu):
    return pl.pallas_call(
        functools.partial(_mm_mid_body, relu=relu),
        grid=(NP // BLR,),
        in_specs=[
            pl.BlockSpec((BLR, D), lambda j: (j, 0)),
            pl.BlockSpec((BLR, D), lambda j: (NP // BLR + j, 0)),
            pl.BlockSpec((BLR, D), lambda j: (j, 0)),
            pl.BlockSpec((BLR, 1), lambda j: (j, 0)),
            pl.BlockSpec((1, D), lambda j: (0, 0)),
            pl.BlockSpec((D, D), lambda j: (0, 0)),
        ],
        out_specs=pl.BlockSpec((BLR, D), lambda j: (j, 0)),
        out_shape=jax.ShapeDtypeStruct((NP, D), jnp.float32),
    )(p, p, g, dis_col, b_row, w_next)


def _mm_final_body(p0_ref, p1_ref, g_ref, dis_ref, b_ref, wl_ref, bl_ref,
                   out_ref):
    x = dis_ref[...] * (p0_ref[...] + p1_ref[...] + g_ref[...]) + b_ref[...]
    out_ref[...] = lax.dot_general(
        x, wl_ref[...], (((1,), (0,)), ((), ())),
        preferred_element_type=jnp.float32) + bl_ref[...]


def _mm_final(p, g, dis_col, b_row, wl, bl_row):
    c = wl.shape[1]
    return pl.pallas_call(
        _mm_final_body,
        grid=(NP // BLR,),
        in_specs=[
            pl.BlockSpec((BLR, D), lambda j: (j, 0)),
            pl.BlockSpec((BLR, D), lambda j: (NP // BLR + j, 0)),
            pl.BlockSpec((BLR, D), lambda j: (j, 0)),
            pl.BlockSpec((BLR, 1), lambda j: (j, 0)),
            pl.BlockSpec((1, D), lambda j: (0, 0)),
            pl.BlockSpec((D, c), lambda j: (0, 0)),
            pl.BlockSpec((1, c), lambda j: (0, 0)),
        ],
        out_specs=pl.BlockSpec((BLR, c), lambda j: (j, 0)),
        out_shape=jax.ShapeDtypeStruct((NP, c), jnp.float32),
    )(p, p, g, dis_col, b_row, wl, bl_row)


def kernel(x, edge_index, W1, b1, W2, b2, W3, b3, Wl, bl):
    src = edge_index[0].astype(jnp.int32)
    dst = edge_index[1].astype(jnp.int32)
    pad = N + (jnp.arange(EP - E, dtype=jnp.int32) % (NP - N))
    src_p = jnp.concatenate([src, pad]).reshape(NW * JROWS, G)
    dst_p = jnp.concatenate([dst, pad]).reshape(NW * JROWS, G)

    x_p = jnp.pad(x, ((0, NP - N), (0, 0)))
    zeros_tile = jnp.zeros((NPT, D), jnp.float32)

    hist = _deg_kernel()(dst).reshape(NW, NP)
    dis_col = _dis_kernel(hist).reshape(NP, 1)

    agg = _agg_kernel()
    g1 = _mm_first(x_p, W1, dis_col)
    p1 = agg(g1, src_p, dst_p, zeros_tile)
    g2 = _mm_mid(p1, g1, dis_col, b1.reshape(1, D), W2, relu=True)
    p2 = agg(g2, src_p, dst_p, zeros_tile)
    g3 = _mm_mid(p2, g2, dis_col, b2.reshape(1, D), W3, relu=True)
    p3 = agg(g3, src_p, dst_p, zeros_tile)
    out = _mm_final(p3, g3, dis_col, b3.reshape(1, D), Wl, bl.reshape(1, -1))
    return out[:N]


# speedup vs baseline: 25.8199x; 1.0011x over previous
import functools

import jax
import jax.numpy as jnp
from jax import lax
from jax.experimental import pallas as pl
from jax.experimental.pallas import tpu as pltpu
from jax.experimental.pallas import tpu_sc as plsc

N = 10000
NP = 10240
D = 128
E = 320000
NC = 2
NS = 16
NW = NC * NS
EPW = E // NW
KBLK = 2000
G = 128
JROWS = 80
CHROWS = 16
EPA = JROWS * G
EP = NW * EPA
NPT = NP // NS
BLR = 1280


def _mesh():
    return plsc.VectorSubcoreMesh(
        core_axis_name="c", subcore_axis_name="s",
        num_cores=NC, num_subcores=NS)


def _deg_body(dst_hbm, out_hbm, d_v, hist_v):
    wid = lax.axis_index("s") * NC + lax.axis_index("c")
    base = wid * EPW
    zeros = jnp.zeros((16,), jnp.float32)
    ones = jnp.ones((16,), jnp.float32)

    def zero_body(i, carry):
        hist_v[pl.ds(i * 16, 16)] = zeros
        return carry
    lax.fori_loop(0, NP // 16, zero_body, 0)

    def blk_body(b, carry):
        pltpu.sync_copy(dst_hbm.at[pl.ds(base + b * KBLK, KBLK)], d_v)

        def grp_body(k, c):
            dv = d_v[pl.ds(k * 16, 16)]
            plsc.addupdate_scatter(hist_v, [dv], ones)
            return c
        lax.fori_loop(0, KBLK // 16, grp_body, 0)
        return carry
    lax.fori_loop(0, EPW // KBLK, blk_body, 0)

    pltpu.sync_copy(hist_v, out_hbm.at[pl.ds(wid * NP, NP)])


@functools.cache
def _deg_kernel():
    return pl.kernel(
        _deg_body,
        out_type=jax.ShapeDtypeStruct((NW * NP,), jnp.float32),
        mesh=_mesh(),
        compiler_params=pltpu.CompilerParams(needs_layout_passes=False),
        scratch_types=[
            pltpu.VMEM((KBLK,), jnp.int32),
            pltpu.VMEM((NP,), jnp.float32),
        ],
    )


def _agg_body(g_hbm, src_hbm, dst_hbm, z_hbm, out_hbm,
              s_idx, d_idx, rows_a, rows_b, acc, sem_a, sem_b,
              ssem_a, ssem_b):
    cid = lax.axis_index("c")
    sid = lax.axis_index("s")
    w2 = cid * NS + sid
    base = w2 * JROWS

    pltpu.sync_copy(z_hbm, acc.at[pl.ds(sid * NPT, NPT)])
    plsc.subcore_barrier()

    bufs = (rows_a, rows_b)
    gsems = (sem_a, sem_b)
    ssems = (ssem_a, ssem_b)
    g_cp = [None, None]
    s_cp = [None, None]
    for c in range(JROWS // CHROWS):
        pltpu.sync_copy(src_hbm.at[pl.ds(base + c * CHROWS, CHROWS)], s_idx)
        pltpu.sync_copy(dst_hbm.at[pl.ds(base + c * CHROWS, CHROWS)], d_idx)
        g_cp[0] = pltpu.async_copy(g_hbm.at[s_idx.at[0]], rows_a, sem_a)
        for j in range(CHROWS):
            cur = j % 2
            nxt = (j + 1) % 2
            if j + 1 < CHROWS:
                if s_cp[nxt] is not None:
                    s_cp[nxt].wait()
                g_cp[nxt] = pltpu.async_copy(
                    g_hbm.at[s_idx.at[j + 1]], bufs[nxt], gsems[nxt])
            g_cp[cur].wait()
            s_cp[cur] = pltpu.async_copy(
                bufs[cur], acc.at[d_idx.at[j]], ssems[cur], add=True)
        for p in range(2):
            if s_cp[p] is not None:
                s_cp[p].wait()
                s_cp[p] = None

    plsc.subcore_barrier()
    pltpu.sync_copy(acc.at[pl.ds(sid * NPT, NPT)],
                    out_hbm.at[pl.ds(cid * NP + sid * NPT, NPT)])


@functools.cache
def _agg_kernel():
    return pl.kernel(
        _agg_body,
        out_type=jax.ShapeDtypeStruct((NC * NP, D), jnp.float32),
        mesh=_mesh(),
        compiler_params=pltpu.CompilerParams(needs_layout_passes=False),
        scratch_types=[
            pltpu.VMEM((CHROWS, G), jnp.int32),
            pltpu.VMEM((CHROWS, G), jnp.int32),
            pltpu.VMEM((G, D), jnp.float32),
            pltpu.VMEM((G, D), jnp.float32),
            pltpu.VMEM_SHARED((NP, D), jnp.float32),
            pltpu.SemaphoreType.DMA,
            pltpu.SemaphoreType.DMA,
            pltpu.SemaphoreType.DMA,
            pltpu.SemaphoreType.DMA,
        ],
    )


def _dis_body(hist_ref, dis_ref):
    deg = jnp.sum(hist_ref[...], axis=0, keepdims=True) + 1.0
    dis_ref[...] = lax.rsqrt(deg)


def _dis_kernel(hist):
    return pl.pallas_call(
        _dis_body,
        grid=(NP // BLR,),
        in_specs=[pl.BlockSpec((NW, BLR), lambda j: (0, j))],
        out_specs=pl.BlockSpec((1, BLR), lambda j: (0, j)),
        out_shape=jax.ShapeDtypeStruct((1, NP), jnp.float32),
    )(hist)


def _mm_first_body(x_ref, w_ref, dis_ref, g_ref):
    h = lax.dot_general(x_ref[...], w_ref[...], (((1,), (0,)), ((), ())),
                        preferred_element_type=jnp.float32)
    g_ref[...] = h * dis_ref[...]


def _mm_first(x_p, w, dis_col):
    return pl.pallas_call(
        _mm_first_body,
        grid=(NP // BLR,),
        in_specs=[
            pl.BlockSpec((BLR, D), lambda j: (j, 0)),
            pl.BlockSpec((D, D), lambda j: (0, 0)),
            pl.BlockSpec((BLR, 1), lambda j: (j, 0)),
        ],
        out_specs=pl.BlockSpec((BLR, D), lambda j: (j, 0)),
        out_shape=jax.ShapeDtypeStruct((NP, D), jnp.float32),
    )(x_p, w, dis_col)


def _mm_mid_body(p0_ref, p1_ref, g_ref, dis_ref, b_ref, w_ref, out_ref, *,
                 relu):
    dis = dis_ref[...]
    x = dis * (p0_ref[...] + p1_ref[...] + g_ref[...]) + b_ref[...]
    if relu:
        x = jnp.maximum(x, 0.0)
    h = lax.dot_general(x, w_ref[...], (((1,), (0,)), ((), ())),
                        preferred_element_type=jnp.float32)
    out_ref[...] = h * dis


def _mm_mid(p, g, dis_col, b_row, w_next, relu):
    return pl.pallas_call(
        functools.partial(_mm_mid_body, relu=relu),
        grid=(NP // BLR,),
        in_specs=[
            pl.BlockSpec((BLR, D), lambda j: (j, 0)),
            pl.BlockSpec((BLR, D), lambda j: (NP // BLR + j, 0)),
            pl.BlockSpec((BLR, D), lambda j: (j, 0)),
            pl.BlockSpec((BLR, 1), lambda j: (j, 0)),
            pl.BlockSpec((1, D), lambda j: (0, 0)),
            pl.BlockSpec((D, D), lambda j: (0, 0)),
        ],
        out_specs=pl.BlockSpec((BLR, D), lambda j: (j, 0)),
        out_shape=jax.ShapeDtypeStruct((NP, D), jnp.float32),
    )(p, p, g, dis_col, b_row, w_next)


def _mm_final_body(p0_ref, p1_ref, g_ref, dis_ref, b_ref, wl_ref, bl_ref,
                   out_ref):
    x = dis_ref[...] * (p0_ref[...] + p1_ref[...] + g_ref[...]) + b_ref[...]
    out_ref[...] = lax.dot_general(
        x, wl_ref[...], (((1,), (0,)), ((), ())),
        preferred_element_type=jnp.float32) + bl_ref[...]


def _mm_final(p, g, dis_col, b_row, wl, bl_row):
    c = wl.shape[1]
    return pl.pallas_call(
        _mm_final_body,
        grid=(NP // BLR,),
        in_specs=[
            pl.BlockSpec((BLR, D), lambda j: (j, 0)),
            pl.BlockSpec((BLR, D), lambda j: (NP // BLR + j, 0)),
            pl.BlockSpec((BLR, D), lambda j: (j, 0)),
            pl.BlockSpec((BLR, 1), lambda j: (j, 0)),
            pl.BlockSpec((1, D), lambda j: (0, 0)),
            pl.BlockSpec((D, c), lambda j: (0, 0)),
            pl.BlockSpec((1, c), lambda j: (0, 0)),
        ],
        out_specs=pl.BlockSpec((BLR, c), lambda j: (j, 0)),
        out_shape=jax.ShapeDtypeStruct((NP, c), jnp.float32),
    )(p, p, g, dis_col, b_row, wl, bl_row)


def kernel(x, edge_index, W1, b1, W2, b2, W3, b3, Wl, bl):
    src = edge_index[0].astype(jnp.int32)
    dst = edge_index[1].astype(jnp.int32)
    pad = N + (jnp.arange(EP - E, dtype=jnp.int32) % (NP - N))
    src_p = jnp.concatenate([src, pad]).reshape(NW * JROWS, G)
    dst_p = jnp.concatenate([dst, pad]).reshape(NW * JROWS, G)

    x_p = jnp.pad(x, ((0, NP - N), (0, 0)))
    zeros_tile = jnp.zeros((NPT, D), jnp.float32)

    hist = _deg_kernel()(dst).reshape(NW, NP)
    dis_col = _dis_kernel(hist).reshape(NP, 1)

    agg = _agg_kernel()
    g1 = _mm_first(x_p, W1, dis_col)
    p1 = agg(g1, src_p, dst_p, zeros_tile)
    g2 = _mm_mid(p1, g1, dis_col, b1.reshape(1, D), W2, relu=True)
    p2 = agg(g2, src_p, dst_p, zeros_tile)
    g3 = _mm_mid(p2, g2, dis_col, b2.reshape(1, D), W3, relu=True)
    p3 = agg(g3, src_p, dst_p, zeros_tile)
    out = _mm_final(p3, g3, dis_col, b3.reshape(1, D), Wl, bl.reshape(1, -1))
    return out[:N]


# speedup vs baseline: 27.2809x; 1.0566x over previous
import functools

import jax
import jax.numpy as jnp
from jax import lax
from jax.experimental import pallas as pl
from jax.experimental.pallas import tpu as pltpu
from jax.experimental.pallas import tpu_sc as plsc

N = 10000
NP = 10240
D = 128
E = 320000
NC = 2
NS = 16
NW = NC * NS
EPW = E // NW
KBLK = 2000
G = 128
JROWS = 80
CHROWS = 40
EPA = JROWS * G
EP = NW * EPA
NPT = NP // NS
BLR = 1280


def _mesh():
    return plsc.VectorSubcoreMesh(
        core_axis_name="c", subcore_axis_name="s",
        num_cores=NC, num_subcores=NS)


def _deg_body(dst_hbm, out_hbm, d_v, hist_v):
    wid = lax.axis_index("s") * NC + lax.axis_index("c")
    base = wid * EPW
    zeros = jnp.zeros((16,), jnp.float32)
    ones = jnp.ones((16,), jnp.float32)

    def zero_body(i, carry):
        hist_v[pl.ds(i * 16, 16)] = zeros
        return carry
    lax.fori_loop(0, NP // 16, zero_body, 0)

    def blk_body(b, carry):
        pltpu.sync_copy(dst_hbm.at[pl.ds(base + b * KBLK, KBLK)], d_v)

        def grp_body(k, c):
            dv = d_v[pl.ds(k * 16, 16)]
            plsc.addupdate_scatter(hist_v, [dv], ones)
            return c
        lax.fori_loop(0, KBLK // 16, grp_body, 0)
        return carry
    lax.fori_loop(0, EPW // KBLK, blk_body, 0)

    pltpu.sync_copy(hist_v, out_hbm.at[pl.ds(wid * NP, NP)])


@functools.cache
def _deg_kernel():
    return pl.kernel(
        _deg_body,
        out_type=jax.ShapeDtypeStruct((NW * NP,), jnp.float32),
        mesh=_mesh(),
        compiler_params=pltpu.CompilerParams(needs_layout_passes=False),
        scratch_types=[
            pltpu.VMEM((KBLK,), jnp.int32),
            pltpu.VMEM((NP,), jnp.float32),
        ],
    )


def _agg_body(g_hbm, src_hbm, dst_hbm, z_hbm, out_hbm,
              s_idx, d_idx, rows_a, rows_b, acc, sem_a, sem_b,
              ssem_a, ssem_b):
    cid = lax.axis_index("c")
    sid = lax.axis_index("s")
    w2 = cid * NS + sid
    base = w2 * JROWS

    pltpu.sync_copy(z_hbm, rows_a)
    for t in range(NPT // G):
        pltpu.sync_copy(rows_a, acc.at[pl.ds(sid * NPT + t * G, G)])
    plsc.subcore_barrier()

    bufs = (rows_a, rows_b)
    gsems = (sem_a, sem_b)
    ssems = (ssem_a, ssem_b)
    g_cp = [None, None]
    s_cp = [None, None]
    for c in range(JROWS // CHROWS):
        pltpu.sync_copy(src_hbm.at[pl.ds(base + c * CHROWS, CHROWS)], s_idx)
        pltpu.sync_copy(dst_hbm.at[pl.ds(base + c * CHROWS, CHROWS)], d_idx)
        g_cp[0] = pltpu.async_copy(g_hbm.at[s_idx.at[0]], rows_a, sem_a)
        for j in range(CHROWS):
            cur = j % 2
            nxt = (j + 1) % 2
            if j + 1 < CHROWS:
                if s_cp[nxt] is not None:
                    s_cp[nxt].wait()
                g_cp[nxt] = pltpu.async_copy(
                    g_hbm.at[s_idx.at[j + 1]], bufs[nxt], gsems[nxt])
            g_cp[cur].wait()
            s_cp[cur] = pltpu.async_copy(
                bufs[cur], acc.at[d_idx.at[j]], ssems[cur], add=True)
        for p in range(2):
            if s_cp[p] is not None:
                s_cp[p].wait()
                s_cp[p] = None

    plsc.subcore_barrier()
    pltpu.sync_copy(acc.at[pl.ds(sid * NPT, NPT)],
                    out_hbm.at[pl.ds(cid * NP + sid * NPT, NPT)])


@functools.cache
def _agg_kernel():
    return pl.kernel(
        _agg_body,
        out_type=jax.ShapeDtypeStruct((NC * NP, D), jnp.float32),
        mesh=_mesh(),
        compiler_params=pltpu.CompilerParams(needs_layout_passes=False),
        scratch_types=[
            pltpu.VMEM((CHROWS, G), jnp.int32),
            pltpu.VMEM((CHROWS, G), jnp.int32),
            pltpu.VMEM((G, D), jnp.float32),
            pltpu.VMEM((G, D), jnp.float32),
            pltpu.VMEM_SHARED((NP, D), jnp.float32),
            pltpu.SemaphoreType.DMA,
            pltpu.SemaphoreType.DMA,
            pltpu.SemaphoreType.DMA,
            pltpu.SemaphoreType.DMA,
        ],
    )


def _dis_body(hist_ref, dis_ref):
    deg = jnp.sum(hist_ref[...], axis=0, keepdims=True) + 1.0
    dis_ref[...] = lax.rsqrt(deg)


def _dis_kernel(hist):
    return pl.pallas_call(
        _dis_body,
        grid=(NP // BLR,),
        in_specs=[pl.BlockSpec((NW, BLR), lambda j: (0, j))],
        out_specs=pl.BlockSpec((1, BLR), lambda j: (0, j)),
        out_shape=jax.ShapeDtypeStruct((1, NP), jnp.float32),
    )(hist)


def _mm_first_body(x_ref, w_ref, dis_ref, g_ref):
    h = lax.dot_general(x_ref[...], w_ref[...], (((1,), (0,)), ((), ())),
                        preferred_element_type=jnp.float32)
    g_ref[...] = h * dis_ref[...]


def _mm_first(x_p, w, dis_col):
    return pl.pallas_call(
        _mm_first_body,
        grid=(NP // BLR,),
        in_specs=[
            pl.BlockSpec((BLR, D), lambda j: (j, 0)),
            pl.BlockSpec((D, D), lambda j: (0, 0)),
            pl.BlockSpec((BLR, 1), lambda j: (j, 0)),
        ],
        out_specs=pl.BlockSpec((BLR, D), lambda j: (j, 0)),
        out_shape=jax.ShapeDtypeStruct((NP, D), jnp.float32),
    )(x_p, w, dis_col)


def _mm_mid_body(p0_ref, p1_ref, g_ref, dis_ref, b_ref, w_ref, out_ref, *,
                 relu):
    dis = dis_ref[...]
    x = dis * (p0_ref[...] + p1_ref[...] + g_ref[...]) + b_ref[...]
    if relu:
        x = jnp.maximum(x, 0.0)
    h = lax.dot_general(x, w_ref[...], (((1,), (0,)), ((), ())),
                        preferred_element_type=jnp.float32)
    out_ref[...] = h * dis


def _mm_mid(p, g, dis_col, b_row, w_next, relu):
    return pl.pallas_call(
        functools.partial(_mm_mid_body, relu=relu),
        grid=(NP // BLR,),
        in_specs=[
            pl.BlockSpec((BLR, D), lambda j: (j, 0)),
            pl.BlockSpec((BLR, D), lambda j: (NP // BLR + j, 0)),
            pl.BlockSpec((BLR, D), lambda j: (j, 0)),
            pl.BlockSpec((BLR, 1), lambda j: (j, 0)),
            pl.BlockSpec((1, D), lambda j: (0, 0)),
            pl.BlockSpec((D, D), lambda j: (0, 0)),
        ],
        out_specs=pl.BlockSpec((BLR, D), lambda j: (j, 0)),
        out_shape=jax.ShapeDtypeStruct((NP, D), jnp.float32),
    )(p, p, g, dis_col, b_row, w_next)


def _mm_final_body(p0_ref, p1_ref, g_ref, dis_ref, b_ref, wl_ref, bl_ref,
                   out_ref):
    x = dis_ref[...] * (p0_ref[...] + p1_ref[...] + g_ref[...]) + b_ref[...]
    out_ref[...] = lax.dot_general(
        x, wl_ref[...], (((1,), (0,)), ((), ())),
        preferred_element_type=jnp.float32) + bl_ref[...]


def _mm_final(p, g, dis_col, b_row, wl, bl_row):
    c = wl.shape[1]
    return pl.pallas_call(
        _mm_final_body,
        grid=(NP // BLR,),
        in_specs=[
            pl.BlockSpec((BLR, D), lambda j: (j, 0)),
            pl.BlockSpec((BLR, D), lambda j: (NP // BLR + j, 0)),
            pl.BlockSpec((BLR, D), lambda j: (j, 0)),
            pl.BlockSpec((BLR, 1), lambda j: (j, 0)),
            pl.BlockSpec((1, D), lambda j: (0, 0)),
            pl.BlockSpec((D, c), lambda j: (0, 0)),
            pl.BlockSpec((1, c), lambda j: (0, 0)),
        ],
        out_specs=pl.BlockSpec((BLR, c), lambda j: (j, 0)),
        out_shape=jax.ShapeDtypeStruct((NP, c), jnp.float32),
    )(p, p, g, dis_col, b_row, wl, bl_row)


def kernel(x, edge_index, W1, b1, W2, b2, W3, b3, Wl, bl):
    src = edge_index[0].astype(jnp.int32)
    dst = edge_index[1].astype(jnp.int32)
    pad = N + (jnp.arange(EP - E, dtype=jnp.int32) % (NP - N))
    src_p = jnp.concatenate([src, pad]).reshape(NW * JROWS, G)
    dst_p = jnp.concatenate([dst, pad]).reshape(NW * JROWS, G)

    x_p = jnp.pad(x, ((0, NP - N), (0, 0)))
    zeros_tile = jnp.zeros((G, D), jnp.float32)

    hist = _deg_kernel()(dst).reshape(NW, NP)
    dis_col = _dis_kernel(hist).reshape(NP, 1)

    agg = _agg_kernel()
    g1 = _mm_first(x_p, W1, dis_col)
    p1 = agg(g1, src_p, dst_p, zeros_tile)
    g2 = _mm_mid(p1, g1, dis_col, b1.reshape(1, D), W2, relu=True)
    p2 = agg(g2, src_p, dst_p, zeros_tile)
    g3 = _mm_mid(p2, g2, dis_col, b2.reshape(1, D), W3, relu=True)
    p3 = agg(g3, src_p, dst_p, zeros_tile)
    out = _mm_final(p3, g3, dis_col, b3.reshape(1, D), Wl, bl.reshape(1, -1))
    return out[:N]
